# Initial kernel scaffold; baseline (speedup 1.0000x reference)
import jax
import jax.numpy as jnp
from jax.experimental import pallas as pl


def kernel(node_features, edge_index, f, W0, al0, ar0, b0, W1, al1, ar1, b1, W2, al2, ar2, b2, W3, al3, ar3, b3, Wf1, bf1, Wf2, bf2, Wf3, bf3):
    raise NotImplementedError("write your pallas kernel here")


# speedup vs baseline: 10.8957x; 10.8957x over previous
import jax
import jax.numpy as jnp
from jax import lax
from jax.experimental import pallas as pl
from jax.experimental.pallas import tpu as pltpu
from jax.experimental.pallas import tpu_sc as plsc

N = 10000
NP = 10240
H = NP // 2
E = 160000
BN = 512
NBLK = NP // BN
L = 16
K = 80
ET = E // 16
NTB = ET // K
FW = 128
ZB = 2560

_f32 = jnp.float32


def _leaky(x):
    return jnp.where(x >= 0, x, 0.1 * x)


def _elu(x):
    return jnp.where(x > 0, x, jnp.exp(jnp.minimum(x, 0.0)) - 1.0)


def _pre0_body(x_ref, w_ref, wa_ref, hst_ref, elerT_ref):
    x = x_ref[...]
    hst_ref[...] = jnp.dot(x, w_ref[0], preferred_element_type=_f32)
    elerT_ref[...] = lax.dot_general(
        wa_ref[...], x, (((0,), (1,)), ((), ())),
        preferred_element_type=_f32)


def _prek_body(a0_ref, a1_ref, d0_ref, d1_ref, p_ref, b_ref, w_ref, wa_ref,
               hst_ref, elerT_ref):
    p = p_ref[...]
    d0 = d0_ref[...]
    d1 = d1_ref[...]
    i0 = 1.0 / jnp.where(d0 > 0, d0, 1.0)
    i1 = 1.0 / jnp.where(d1 > 0, d1, 1.0)
    b = b_ref[...]
    x0 = _elu(a0_ref[...] * jnp.dot(i0, p, preferred_element_type=_f32)
              + b[:, :128])
    x1 = _elu(a1_ref[...] * jnp.dot(i1, p, preferred_element_type=_f32)
              + b[:, 128:])
    x = jnp.concatenate([x0, x1], axis=1)
    hst_ref[...] = jnp.dot(x, w_ref[0], preferred_element_type=_f32)
    elerT_ref[...] = lax.dot_general(
        wa_ref[...], x, (((0,), (1,)), ((), ())),
        preferred_element_type=_f32)


def _tc_pre(layer0, nsp, xin, a_prev, d_prev, p4, b_prev, w, wa):
    din = w.shape[0]
    grid = (NBLK, nsp)
    out_shape = [
        jax.ShapeDtypeStruct((nsp * NP, FW), _f32),
        jax.ShapeDtypeStruct((16, NP), _f32),
    ]
    out_specs = [
        pl.BlockSpec((BN, FW), lambda i, h: (h * NBLK + i, 0)),
        pl.BlockSpec((16, BN), lambda i, h: (0, i)),
    ]
    w_spec = pl.BlockSpec((1, din, FW), lambda i, h: (h, 0, 0))
    wst = jnp.stack([w[:, j * FW:(j + 1) * FW] for j in range(nsp)])
    wa_spec = pl.BlockSpec((din, 16), lambda i, h: (0, 0))
    if layer0:
        in_specs = [
            pl.BlockSpec((BN, din), lambda i, h: (i, 0)),
            w_spec, wa_spec,
        ]
        return pl.pallas_call(
            _pre0_body, grid=grid, in_specs=in_specs, out_specs=out_specs,
            out_shape=out_shape)(xin, wst, wa)
    in_specs = [
        pl.BlockSpec((BN, FW), lambda i, h: (i, 0)),
        pl.BlockSpec((BN, FW), lambda i, h: (NBLK + i, 0)),
        pl.BlockSpec((BN, 4), lambda i, h: (i, 0)),
        pl.BlockSpec((BN, 4), lambda i, h: (NBLK + i, 0)),
        pl.BlockSpec((4, 128), lambda i, h: (0, 0)),
        pl.BlockSpec((1, 256), lambda i, h: (0, 0)),
        w_spec, wa_spec,
    ]
    return pl.pallas_call(
        _prek_body, grid=grid, in_specs=in_specs, out_specs=out_specs,
        out_shape=out_shape)(a_prev, a_prev, d_prev, d_prev, p4,
                             b_prev.reshape(1, 256), wst, wa)


def _mlp_body(a3_ref, d3_ref, b3_ref, f_ref, wf1_ref, bf1_ref,
              wf2_ref, bf2_ref, wf3_ref, bf3_ref, out_ref, acc_ref):
    k = pl.program_id(0)

    @pl.when(k == 0)
    def _():
        acc_ref[...] = jnp.zeros_like(acc_ref)

    @pl.when(k < 100)
    def _():
        h = a3_ref[...].reshape(100, 128)
        d = d3_ref[...].reshape(100, 4)[:, 0:1]
        h3 = h * (1.0 / jnp.where(d > 0, d, 1.0)) + b3_ref[...]
        acc_ref[...] += jnp.dot(h3, wf1_ref[...], preferred_element_type=_f32)

    @pl.when(k == 100)
    def _():
        acc_ref[...] += jnp.dot(f_ref[...], wf1_ref[...],
                                preferred_element_type=_f32)
        z1 = jnp.maximum(acc_ref[...] + bf1_ref[...], 0.0)
        z2 = jnp.maximum(
            jnp.dot(z1, wf2_ref[...], preferred_element_type=_f32)
            + bf2_ref[...], 0.0)
        out_ref[...] = (jnp.dot(z2, wf3_ref[...], preferred_element_type=_f32)
                        + bf3_ref[...])


def _tc_mlp(a3, d3, b3, f, wf1, bf1, wf2, bf2, wf3, bf3):
    grid = (101,)
    cl = lambda k: (0, jnp.minimum(k, 99), 0, 0)
    in_specs = [
        pl.BlockSpec((100, 1, 1, 128), cl),
        pl.BlockSpec((100, 1, 1, 4), cl),
        pl.BlockSpec((1, 128), lambda k: (0, 0)),
        pl.BlockSpec((100, 128), lambda k: (0, 0)),
        pl.BlockSpec((128, 500), lambda k: (k, 0)),
        pl.BlockSpec((1, 500), lambda k: (0, 0)),
        pl.BlockSpec((500, 500), lambda k: (0, 0)),
        pl.BlockSpec((1, 500), lambda k: (0, 0)),
        pl.BlockSpec((500, 1), lambda k: (0, 0)),
        pl.BlockSpec((1, 1), lambda k: (0, 0)),
    ]
    return pl.pallas_call(
        _mlp_body, grid=grid, in_specs=in_specs,
        out_specs=pl.BlockSpec((100, 1), lambda k: (0, 0)),
        out_shape=jax.ShapeDtypeStruct((100, 1), _f32),
        scratch_shapes=[pltpu.VMEM((100, 500), _f32)],
    )(a3, d3, b3, f, wf1, bf1, wf2, bf2, wf3, bf3)


def _make_edge_kernel(hl, l3):
    npass = 1 if l3 else 2
    cph = (FW // L) // hl
    grp = K // L

    def body(hst, elerT, src, dst, agg_out, den_out,
             eltab, srcb, dstb, rows, wrow, denl, idxb, eridx, erbuf,
             sem, agg_sp, den_sp):
        cid = lax.axis_index("c")
        sid = lax.axis_index("s")

        el_base = 0 if l3 else cid * hl
        er_base = 8 + el_base
        pltpu.sync_copy(elerT.at[pl.ds(el_base * NP, hl * NP)], eltab)

        lane = lax.iota(jnp.int32, L)
        zvec = lax.convert_element_type(lane, _f32) * 0.0

        DR = H * 4 // 128

        def idf(r, _):
            idxb[pl.ds(r * L, L)] = lane + r * L
            return 0
        lax.fori_loop(0, DR // L, idf, 0)

        elmax = []
        for hd in range(hl):
            def mx(i, m):
                return jnp.maximum(m, eltab[pl.ds(hd * NP + i * L, L)])
            m = lax.fori_loop(0, NP // L, mx, zvec - 3e38)
            for sh in (8, 4, 2, 1):
                perm = (lane + sh) & (L - 1)
                rot = lax.gather(
                    m, perm[:, None],
                    lax.GatherDimensionNumbers(
                        offset_dims=(), collapsed_slice_dims=(0,),
                        start_index_map=(0,)),
                    slice_sizes=(1,),
                    mode=lax.GatherScatterMode.PROMISE_IN_BOUNDS)
                m = jnp.maximum(m, rot)
            elmax.append(m)

        ebase = sid * ET
        roff = 0 if l3 else cid * NP
        rpt = H // 16

        for q in range(npass):
            dst_base = cid * H if l3 else q * H

            def zro(r, _):
                for c in range(FW // L):
                    rows[r, pl.ds(c * L, L)] = zvec
                return 0
            lax.fori_loop(0, K, zro, 0)

            def zro3(r, _):
                for c in range(128 // L):
                    denl[r, pl.ds(c * L, L)] = zvec
                return 0
            lax.fori_loop(0, DR, zro3, 0)

            for j in range(rpt // K):
                pltpu.sync_copy(
                    rows,
                    agg_sp.at[pl.ds(pl.multiple_of(sid * rpt + j * K, 8),
                                    K)])
            @pl.when(sid == 0)
            def _():
                pltpu.sync_copy(rows.at[pl.ds(0, 8)],
                                agg_sp.at[pl.ds(H, 8)])
            @pl.when(sid < 4)
            def _():
                pltpu.sync_copy(
                    rows.at[pl.ds(0, DR // 4)],
                    den_sp.at[pl.ds(pl.multiple_of(sid * (DR // 4), 8),
                                    DR // 4)])
            plsc.subcore_barrier()

            def block(ib, _):
                off = ebase + ib * K
                pltpu.sync_copy(src.at[pl.ds(off, K)], srcb)
                pltpu.sync_copy(dst.at[pl.ds(off, K)], dstb)

                for g in range(grp):
                    d16g = dstb[pl.ds(g * L, L)]
                    for j in range(hl):
                        eridx[pl.ds(j * K + g * L, L)] = (
                            d16g + (er_base + j) * NP)
                descs = [
                    pltpu.async_copy(
                        elerT.at[eridx.at[pl.ds(j * K, K)]],
                        erbuf.at[pl.ds(j * K, K)], sem)
                    for j in range(hl)
                ]
                for d in descs:
                    d.wait()

                for g in range(grp):
                    s16 = srcb[pl.ds(g * L, L)]
                    d16 = dstb[pl.ds(g * L, L)]
                    ld = d16 - dst_base
                    msk = (ld >= 0) & (ld < H)
                    ldc = jnp.where(msk, ld, 0)
                    for j in range(hl):
                        el_s = plsc.load_gather(
                            eltab, [lane * 0 + j * NP + s16])
                        er_d = erbuf[pl.ds(j * K + g * L, L)]
                        e = _leaky(el_s + er_d)
                        bnd = _leaky(elmax[j] + er_d)
                        w = jnp.exp(e - bnd)
                        wrow[pl.ds(j * K + g * L, L)] = w
                        p = ldc * 4 + j
                        plsc.addupdate_scatter(denl, [p >> 7, p & 127], w,
                                               mask=msk)
                    dstb[pl.ds(g * L, L)] = jnp.where(msk, ld, H)
                    srcb[pl.ds(g * L, L)] = s16 + roff

                pltpu.sync_copy(hst.at[srcb], rows)

                def scale(e, _):
                    esp = lane * 0 + e
                    for j in range(hl):
                        wv = plsc.load_gather(wrow, [esp + j * K])
                        for cc in range(cph):
                            c = j * cph + cc
                            rows[e, pl.ds(c * L, L)] = (
                                rows[e, pl.ds(c * L, L)] * wv)
                    return 0
                lax.fori_loop(0, K, scale, 0)

                pltpu.sync_copy(rows, agg_sp.at[dstb], add=True)
                return 0

            lax.fori_loop(0, NTB, block, 0)

            pltpu.sync_copy(denl, den_sp.at[idxb], add=True)
            plsc.subcore_barrier()

            out_base = roff + dst_base
            pltpu.sync_copy(
                agg_sp.at[pl.ds(pl.multiple_of(sid * rpt, 8), rpt)],
                agg_out.at[pl.ds(pl.multiple_of(out_base + sid * rpt, 8),
                                 rpt)])
            @pl.when(sid < 4)
            def _():
                pltpu.sync_copy(
                    den_sp.at[pl.ds(pl.multiple_of(sid * (DR // 4), 8),
                                    DR // 4)],
                    den_out.at[pl.ds(
                        pl.multiple_of(out_base // 32 + sid * (DR // 4),
                                       8), DR // 4)])
            if q != npass - 1:
                plsc.subcore_barrier()

    mesh = plsc.VectorSubcoreMesh(core_axis_name="c", subcore_axis_name="s",
                                  num_cores=2, num_subcores=16)
    nst = 1 if l3 else 2
    return pl.kernel(
        body,
        out_type=[
            jax.ShapeDtypeStruct((nst * NP, FW), _f32),
            jax.ShapeDtypeStruct((nst * NP * 4 // 128, 128), _f32),
        ],
        mesh=mesh,
        compiler_params=pltpu.CompilerParams(needs_layout_passes=False),
        scratch_types=[
            pltpu.VMEM((hl * NP,), _f32),
            pltpu.VMEM((K,), jnp.int32),
            pltpu.VMEM((K,), jnp.int32),
            pltpu.VMEM((K, FW), _f32),
            pltpu.VMEM((hl * K,), _f32),
            pltpu.VMEM((H * 4 // 128, 128), _f32),
            pltpu.VMEM((H * 4 // 128,), jnp.int32),
            pltpu.VMEM((hl * K,), jnp.int32),
            pltpu.VMEM((hl * K,), _f32),
            pltpu.SemaphoreType.DMA,
            pltpu.VMEM_SHARED((H + 8, FW), _f32),
            pltpu.VMEM_SHARED((H * 4 // 128, 128), _f32),
        ],
    )


def _wa(w, al, ar, heads, dout):
    wr = w.reshape(w.shape[0], heads, dout)
    wal = jnp.einsum("dhj,hj->dh", wr, al)
    war = jnp.einsum("dhj,hj->dh", wr, ar)
    pad = jnp.zeros((w.shape[0], 8 - heads), _f32)
    return jnp.concatenate([wal, pad, war, pad], axis=1)


def kernel(node_features, edge_index, f, W0, al0, ar0, b0, W1, al1, ar1, b1,
           W2, al2, ar2, b2, W3, al3, ar3, b3, Wf1, bf1, Wf2, bf2, Wf3, bf3):
    src = edge_index[0].astype(jnp.int32)
    dst = edge_index[1].astype(jnp.int32)

    p4 = jnp.zeros((4, 128), _f32).at[
        jnp.repeat(jnp.arange(4), 32), jnp.arange(128)].set(1.0)

    edge_k = _make_edge_kernel(4, False)

    w3z = jnp.concatenate([W3, jnp.zeros((256, 128), _f32)], axis=1)
    wal3 = (W3 @ al3[0])[:, None]
    war3 = (W3 @ ar3[0])[:, None]
    wa3 = jnp.concatenate([jnp.tile(wal3, (1, 8)),
                           jnp.tile(war3, (1, 8))], axis=1)

    xpad = jnp.pad(node_features, ((0, NP - N), (0, 0)))
    hst, elerT = _tc_pre(True, 2, xpad, None, None, None, None,
                         W0, _wa(W0, al0, ar0, 8, 32))

    def mk_pre(b_prev, w, wa):
        def go(agg, den4):
            h, e = _tc_pre(False, 2, None, agg, den4, p4, b_prev, w, wa)
            return h, e.reshape(-1)
        return go

    branches = [
        mk_pre(b0, W1, _wa(W1, al1, ar1, 8, 32)),
        mk_pre(b1, W2, _wa(W2, al2, ar2, 8, 32)),
        mk_pre(b2, w3z, wa3),
    ]

    def body(i, carry):
        hst_c, eler_c, _, _ = carry
        agg, den = edge_k(hst_c, eler_c, src, dst)
        den_flat = den.reshape(-1)
        idle = lambda a, d: (hst_c, eler_c)
        hst_n, eler_n = lax.switch(i, branches + [idle], agg,
                                   den_flat.reshape(2 * NP, 4))
        return (hst_n, eler_n, agg, den_flat)

    carry = (hst, elerT.reshape(-1),
             jnp.zeros((2 * NP, FW), _f32), jnp.zeros((2 * NP * 4,), _f32))
    carry = lax.fori_loop(0, 4, body, carry)
    agg3, den3 = carry[2], carry[3].reshape(2 * NP, 4)

    a3 = agg3[:N].reshape(100, 100, 1, 128)
    d3 = den3[:N].reshape(100, 100, 1, 4)
    out = _tc_mlp(a3, d3, b3.reshape(1, 128), f, Wf1,
                  bf1.reshape(1, 500), Wf2, bf2.reshape(1, 500),
                  Wf3, bf3.reshape(1, 1))
    return out.reshape(-1)


# speedup vs baseline: 17.6846x; 1.6231x over previous
import jax
import jax.numpy as jnp
from jax import lax
from jax.experimental import pallas as pl
from jax.experimental.pallas import tpu as pltpu
from jax.experimental.pallas import tpu_sc as plsc

N = 10000
NP = 10240
H = NP // 2
E = 160000
BN = 512
NBLK = NP // BN
L = 16
K = 80
ET = E // 16
NTB = ET // K
FW = 128

_f32 = jnp.float32


def _leaky(x):
    return jnp.where(x >= 0, x, 0.1 * x)


def _elu(x):
    return jnp.where(x > 0, x, jnp.exp(jnp.minimum(x, 0.0)) - 1.0)


def _pre0_body(x_ref, w_ref, wa_ref, hst_ref, elerT_ref):
    x = x_ref[...]
    hst_ref[...] = jnp.dot(x, w_ref[0], preferred_element_type=_f32)
    elerT_ref[...] = lax.dot_general(
        wa_ref[...], x, (((0,), (1,)), ((), ())),
        preferred_element_type=_f32)


def _prek_body(a0_ref, a1_ref, d0_ref, d1_ref, p_ref, b_ref, w_ref, wa_ref,
               hst_ref, elerT_ref):
    p = p_ref[...]
    d0 = d0_ref[...]
    d1 = d1_ref[...]
    i0 = 1.0 / jnp.where(d0 > 0, d0, 1.0)
    i1 = 1.0 / jnp.where(d1 > 0, d1, 1.0)
    b = b_ref[...]
    x0 = _elu(a0_ref[...] * jnp.dot(i0, p, preferred_element_type=_f32)
              + b[:, :128])
    x1 = _elu(a1_ref[...] * jnp.dot(i1, p, preferred_element_type=_f32)
              + b[:, 128:])
    x = jnp.concatenate([x0, x1], axis=1)
    hst_ref[...] = jnp.dot(x, w_ref[0], preferred_element_type=_f32)
    elerT_ref[...] = lax.dot_general(
        wa_ref[...], x, (((0,), (1,)), ((), ())),
        preferred_element_type=_f32)


def _tc_pre(layer0, nsp, xin, a_prev, d_prev, p4, b_prev, w, wa):
    din = w.shape[0]
    grid = (NBLK, nsp)
    out_shape = [
        jax.ShapeDtypeStruct((nsp * NP, FW), _f32),
        jax.ShapeDtypeStruct((16, NP), _f32),
    ]
    out_specs = [
        pl.BlockSpec((BN, FW), lambda i, h: (h * NBLK + i, 0)),
        pl.BlockSpec((16, BN), lambda i, h: (0, i)),
    ]
    w_spec = pl.BlockSpec((1, din, FW), lambda i, h: (h, 0, 0))
    wst = jnp.stack([w[:, j * FW:(j + 1) * FW] for j in range(nsp)])
    wa_spec = pl.BlockSpec((din, 16), lambda i, h: (0, 0))
    if layer0:
        in_specs = [
            pl.BlockSpec((BN, din), lambda i, h: (i, 0)),
            w_spec, wa_spec,
        ]
        return pl.pallas_call(
            _pre0_body, grid=grid, in_specs=in_specs, out_specs=out_specs,
            out_shape=out_shape)(xin, wst, wa)
    in_specs = [
        pl.BlockSpec((BN, FW), lambda i, h: (i, 0)),
        pl.BlockSpec((BN, FW), lambda i, h: (NBLK + i, 0)),
        pl.BlockSpec((BN, 4), lambda i, h: (i, 0)),
        pl.BlockSpec((BN, 4), lambda i, h: (NBLK + i, 0)),
        pl.BlockSpec((4, 128), lambda i, h: (0, 0)),
        pl.BlockSpec((1, 256), lambda i, h: (0, 0)),
        w_spec, wa_spec,
    ]
    return pl.pallas_call(
        _prek_body, grid=grid, in_specs=in_specs, out_specs=out_specs,
        out_shape=out_shape)(a_prev, a_prev, d_prev, d_prev, p4,
                             b_prev.reshape(1, 256), wst, wa)


def _mlp_body(a3_ref, d3_ref, b3_ref, f_ref, wf1_ref, bf1_ref,
              wf2_ref, bf2_ref, wf3_ref, bf3_ref, out_ref, acc_ref):
    k = pl.program_id(0)

    @pl.when(k == 0)
    def _():
        acc_ref[...] = jnp.zeros_like(acc_ref)

    @pl.when(k < 100)
    def _():
        h = a3_ref[...].reshape(100, 128)
        d = d3_ref[...].reshape(100, 4)[:, 0:1]
        h3 = h * (1.0 / jnp.where(d > 0, d, 1.0)) + b3_ref[...]
        acc_ref[...] += jnp.dot(h3, wf1_ref[...], preferred_element_type=_f32)

    @pl.when(k == 100)
    def _():
        acc_ref[...] += jnp.dot(f_ref[...], wf1_ref[...],
                                preferred_element_type=_f32)
        z1 = jnp.maximum(acc_ref[...] + bf1_ref[...], 0.0)
        z2 = jnp.maximum(
            jnp.dot(z1, wf2_ref[...], preferred_element_type=_f32)
            + bf2_ref[...], 0.0)
        out_ref[...] = (jnp.dot(z2, wf3_ref[...], preferred_element_type=_f32)
                        + bf3_ref[...])


def _tc_mlp(a3, d3, b3, f, wf1, bf1, wf2, bf2, wf3, bf3):
    grid = (101,)
    cl = lambda k: (0, jnp.minimum(k, 99), 0, 0)
    in_specs = [
        pl.BlockSpec((100, 1, 1, 128), cl),
        pl.BlockSpec((100, 1, 1, 4), cl),
        pl.BlockSpec((1, 128), lambda k: (0, 0)),
        pl.BlockSpec((100, 128), lambda k: (0, 0)),
        pl.BlockSpec((128, 500), lambda k: (k, 0)),
        pl.BlockSpec((1, 500), lambda k: (0, 0)),
        pl.BlockSpec((500, 500), lambda k: (0, 0)),
        pl.BlockSpec((1, 500), lambda k: (0, 0)),
        pl.BlockSpec((500, 1), lambda k: (0, 0)),
        pl.BlockSpec((1, 1), lambda k: (0, 0)),
    ]
    return pl.pallas_call(
        _mlp_body, grid=grid, in_specs=in_specs,
        out_specs=pl.BlockSpec((100, 1), lambda k: (0, 0)),
        out_shape=jax.ShapeDtypeStruct((100, 1), _f32),
        scratch_shapes=[pltpu.VMEM((100, 500), _f32)],
    )(a3, d3, b3, f, wf1, bf1, wf2, bf2, wf3, bf3)


def _make_edge_kernel(hl, l3):
    npass = 1 if l3 else 2
    cph = (FW // L) // hl
    grp = K // L

    def body(hst, elerT, src, dst, agg_out, den_out,
             eltab, erwin, wrow,
             srcb0, srcb1, dstb0, dstb1, mskb0, mskb1, rows0, rows1,
             semR0, semR1, semS0, semS1, semD,
             agg_sp, den_sp):
        cid = lax.axis_index("c")
        sid = lax.axis_index("s")
        srcb = (srcb0, srcb1)
        dstb = (dstb0, dstb1)
        mskb = (mskb0, mskb1)
        rows = (rows0, rows1)
        semR = (semR0, semR1)
        semS = (semS0, semS1)

        el_base = 0 if l3 else cid * hl
        er_base = 8 + el_base
        for j in range(hl):
            pltpu.sync_copy(elerT.at[pl.ds((el_base + j) * NP, N)],
                            eltab.at[pl.ds(j * N, N)])

        lane = lax.iota(jnp.int32, L)
        zvec = lax.convert_element_type(lane, _f32) * 0.0

        elmax = []
        for hd in range(hl):
            def mx(i, m):
                return jnp.maximum(m, eltab[pl.ds(hd * N + i * L, L)])
            m = lax.fori_loop(0, N // L, mx, zvec - 3e38)
            for sh in (8, 4, 2, 1):
                perm = (lane + sh) & (L - 1)
                rot = lax.gather(
                    m, perm[:, None],
                    lax.GatherDimensionNumbers(
                        offset_dims=(), collapsed_slice_dims=(0,),
                        start_index_map=(0,)),
                    slice_sizes=(1,),
                    mode=lax.GatherScatterMode.PROMISE_IN_BOUNDS)
                m = jnp.maximum(m, rot)
            elmax.append(m)

        ebase = sid * ET
        roff = 0 if l3 else cid * NP
        rpt = H // 16
        DW = H * hl

        for q in range(npass):
            dst_base = cid * H if l3 else q * H

            def zro(r, _):
                for c in range(FW // L):
                    rows0[r, pl.ds(c * L, L)] = zvec
                return 0
            lax.fori_loop(0, K, zro, 0)

            def zro2(r, _):
                erwin[pl.ds(r * L, L)] = zvec
                return 0
            lax.fori_loop(0, DW // 16 // L, zro2, 0)

            for j in range(rpt // K):
                pltpu.sync_copy(
                    rows0,
                    agg_sp.at[pl.ds(pl.multiple_of(sid * rpt + j * K, 8),
                                    K)])
            pltpu.sync_copy(
                erwin.at[pl.ds(0, DW // 16)],
                den_sp.at[pl.ds(pl.multiple_of(sid * (DW // 16), 8),
                                DW // 16)])

            for j in range(hl):
                pltpu.sync_copy(
                    elerT.at[pl.ds((er_base + j) * NP + dst_base, H)],
                    erwin.at[pl.ds(j * H, H)])
            plsc.subcore_barrier()

            def start(ib, p, wait_scatter):
                off = ebase + ib * K
                pltpu.sync_copy(src.at[pl.ds(off, K)], srcb[p])
                pltpu.sync_copy(dst.at[pl.ds(off, K)], dstb[p])
                for g in range(grp):
                    s16 = srcb[p][pl.ds(g * L, L)]
                    d16 = dstb[p][pl.ds(g * L, L)]
                    ld = d16 - dst_base
                    msk = (ld >= 0) & (ld < H)
                    mskb[p][pl.ds(g * L, L)] = jnp.where(msk, 1.0, 0.0)
                    dstb[p][pl.ds(g * L, L)] = jnp.where(msk, ld, 0)
                    srcb[p][pl.ds(g * L, L)] = s16 + roff
                if wait_scatter is None:
                    pass
                elif wait_scatter is True:
                    pltpu.make_async_copy(rows[p], agg_sp.at[dstb[p]],
                                          semS[p]).wait()
                else:
                    @pl.when(wait_scatter)
                    def _():
                        pltpu.make_async_copy(rows[p],
                                              agg_sp.at[dstb[p]],
                                              semS[p]).wait()
                pltpu.async_copy(hst.at[srcb[p]], rows[p], semR[p])

            def process(ib, p):
                for g in range(grp):
                    s16a = srcb[p][pl.ds(g * L, L)]
                    ld16 = dstb[p][pl.ds(g * L, L)]
                    mskf = mskb[p][pl.ds(g * L, L)]
                    for j in range(hl):
                        el_s = plsc.load_gather(
                            eltab, [s16a + (j * N - roff)])
                        er_d = plsc.load_gather(erwin, [ld16 + j * H])
                        e = _leaky(el_s + er_d)
                        bnd = _leaky(elmax[j] + er_d)
                        w = jnp.exp(e - bnd) * mskf
                        wrow[pl.ds(j * K + g * L, L)] = w
                        pltpu.async_copy(
                            wrow.at[pl.ds(j * K + g * L, L)],
                            den_sp.at[ld16 * hl + j], semD, add=True)
                pltpu.make_async_copy(hst.at[srcb[p]], rows[p],
                                     semR[p]).wait()

                def scale(e, _):
                    esp = lane * 0 + e
                    for j in range(hl):
                        wv = plsc.load_gather(wrow, [esp + j * K])
                        for cc in range(cph):
                            c = j * cph + cc
                            rows[p][e, pl.ds(c * L, L)] = (
                                rows[p][e, pl.ds(c * L, L)] * wv)
                    return 0
                lax.fori_loop(0, K, scale, 0)

                for g in range(grp):
                    for j in range(hl):
                        pltpu.make_async_copy(
                            wrow.at[pl.ds(j * K + g * L, L)],
                            den_sp.at[lane], semD).wait()

                pltpu.async_copy(rows[p], agg_sp.at[dstb[p]], semS[p],
                                 add=True)

            start(0, 0, None)

            def dbl(t, _):
                start(2 * t + 1, 1, t > 0)
                process(2 * t, 0)
                start(2 * t + 2, 0, True)
                process(2 * t + 1, 1)
                return 0
            lax.fori_loop(0, (NTB - 1) // 2, dbl, 0)
            process(NTB - 1, 0)
            pltpu.make_async_copy(rows[0], agg_sp.at[dstb[0]],
                                  semS[0]).wait()
            pltpu.make_async_copy(rows[1], agg_sp.at[dstb[1]],
                                  semS[1]).wait()
            plsc.subcore_barrier()

            out_base = roff + dst_base
            pltpu.sync_copy(
                agg_sp.at[pl.ds(pl.multiple_of(sid * rpt, 8), rpt)],
                agg_out.at[pl.ds(pl.multiple_of(out_base + sid * rpt, 8),
                                 rpt)])
            pltpu.sync_copy(
                den_sp.at[pl.ds(pl.multiple_of(sid * (DW // 16), 8),
                                DW // 16)],
                den_out.at[pl.ds(
                    pl.multiple_of(out_base * hl + sid * (DW // 16), 8),
                    DW // 16)])
            if q != npass - 1:
                plsc.subcore_barrier()

    mesh = plsc.VectorSubcoreMesh(core_axis_name="c", subcore_axis_name="s",
                                  num_cores=2, num_subcores=16)
    return pl.kernel(
        body,
        out_type=[
            jax.ShapeDtypeStruct((2 * NP, FW), _f32),
            jax.ShapeDtypeStruct((2 * NP * 4,), _f32),
        ],
        mesh=mesh,
        compiler_params=pltpu.CompilerParams(needs_layout_passes=False),
        scratch_types=[
            pltpu.VMEM((hl * N,), _f32),
            pltpu.VMEM((hl * H,), _f32),
            pltpu.VMEM((hl * K,), _f32),
            pltpu.VMEM((K,), jnp.int32),
            pltpu.VMEM((K,), jnp.int32),
            pltpu.VMEM((K,), jnp.int32),
            pltpu.VMEM((K,), jnp.int32),
            pltpu.VMEM((K,), _f32),
            pltpu.VMEM((K,), _f32),
            pltpu.VMEM((K, FW), _f32),
            pltpu.VMEM((K, FW), _f32),
            pltpu.SemaphoreType.DMA,
            pltpu.SemaphoreType.DMA,
            pltpu.SemaphoreType.DMA,
            pltpu.SemaphoreType.DMA,
            pltpu.SemaphoreType.DMA,
            pltpu.VMEM_SHARED((H, FW), _f32),
            pltpu.VMEM_SHARED((H * 4,), _f32),
        ],
    )


def _wa(w, al, ar, heads, dout):
    wr = w.reshape(w.shape[0], heads, dout)
    wal = jnp.einsum("dhj,hj->dh", wr, al)
    war = jnp.einsum("dhj,hj->dh", wr, ar)
    pad = jnp.zeros((w.shape[0], 8 - heads), _f32)
    return jnp.concatenate([wal, pad, war, pad], axis=1)


def kernel(node_features, edge_index, f, W0, al0, ar0, b0, W1, al1, ar1, b1,
           W2, al2, ar2, b2, W3, al3, ar3, b3, Wf1, bf1, Wf2, bf2, Wf3, bf3):
    src = edge_index[0].astype(jnp.int32)
    dst = edge_index[1].astype(jnp.int32)

    p4 = jnp.zeros((4, 128), _f32).at[
        jnp.repeat(jnp.arange(4), 32), jnp.arange(128)].set(1.0)

    edge_k = _make_edge_kernel(4, False)

    w3z = jnp.concatenate([W3, jnp.zeros((256, 128), _f32)], axis=1)
    wal3 = (W3 @ al3[0])[:, None]
    war3 = (W3 @ ar3[0])[:, None]
    wa3 = jnp.concatenate([jnp.tile(wal3, (1, 8)),
                           jnp.tile(war3, (1, 8))], axis=1)

    xpad = jnp.pad(node_features, ((0, NP - N), (0, 0)))
    hst, elerT = _tc_pre(True, 2, xpad, None, None, None, None,
                         W0, _wa(W0, al0, ar0, 8, 32))

    def mk_pre(b_prev, w, wa):
        def go(agg, den4):
            h, e = _tc_pre(False, 2, None, agg, den4, p4, b_prev, w, wa)
            return h, e.reshape(-1)
        return go

    branches = [
        mk_pre(b0, W1, _wa(W1, al1, ar1, 8, 32)),
        mk_pre(b1, W2, _wa(W2, al2, ar2, 8, 32)),
        mk_pre(b2, w3z, wa3),
    ]

    def body(i, carry):
        hst_c, eler_c, _, _ = carry
        agg, den = edge_k(hst_c, eler_c, src, dst)
        den_flat = den.reshape(-1)
        idle = lambda a, d: (hst_c, eler_c)
        hst_n, eler_n = lax.switch(i, branches + [idle], agg,
                                   den_flat.reshape(2 * NP, 4))
        return (hst_n, eler_n, agg, den_flat)

    carry = (hst, elerT.reshape(-1),
             jnp.zeros((2 * NP, FW), _f32), jnp.zeros((2 * NP * 4,), _f32))
    carry = lax.fori_loop(0, 4, body, carry)
    agg3, den3 = carry[2], carry[3].reshape(2 * NP, 4)

    a3 = agg3[:N].reshape(100, 100, 1, 128)
    d3 = den3[:N].reshape(100, 100, 1, 4)
    out = _tc_mlp(a3, d3, b3.reshape(1, 128), f, Wf1,
                  bf1.reshape(1, 500), Wf2, bf2.reshape(1, 500),
                  Wf3, bf3.reshape(1, 1))
    return out.reshape(-1)
